# Initial kernel scaffold; baseline (speedup 1.0000x reference)
import jax
import jax.numpy as jnp
from jax.experimental import pallas as pl


def kernel(node_features, edge_indices, W, b):
    raise NotImplementedError("write your pallas kernel here")


# speedup vs baseline: 12.8369x; 12.8369x over previous
import functools

import jax
import jax.numpy as jnp
from jax import lax
from jax.experimental import pallas as pl
from jax.experimental.pallas import tpu as pltpu
from jax.experimental.pallas import tpu_sc as plsc


def _sc_segment_sum(n_pad, d, nw, n_chunks, k):
    mesh = plsc.VectorSubcoreMesh(core_axis_name="c", subcore_axis_name="s")
    rows_per_sub = n_pad // 16

    @functools.partial(
        pl.kernel,
        out_type=[
            jax.ShapeDtypeStruct((n_pad, d), jnp.float32),
            jax.ShapeDtypeStruct((n_pad, d), jnp.float32),
            jax.ShapeDtypeStruct((n_pad,), jnp.float32),
            jax.ShapeDtypeStruct((n_pad,), jnp.float32),
        ],
        mesh=mesh,
        scratch_types=[
            pltpu.VMEM((n_chunks, k), jnp.int32),
            pltpu.VMEM((n_chunks, k), jnp.int32),
            pltpu.VMEM((k, d), jnp.float32),
            pltpu.VMEM((k,), jnp.float32),
            pltpu.VMEM_SHARED((n_pad, d), jnp.float32),
            pltpu.VMEM_SHARED((n_pad,), jnp.float32),
            pltpu.SemaphoreType.DMA,
        ],
    )
    def sc_kernel(src_hbm, dst_hbm, table_hbm,
                  psum0, psum1, pcnt0, pcnt1,
                  src_v, dst_v, rows_v, ones_v, acc_s, cnt_s, sem):
        cid = lax.axis_index("c")
        sid = lax.axis_index("s")
        wid = sid * 2 + cid

        zeros16 = jnp.zeros((16,), jnp.float32)
        ones16 = jnp.ones((16,), jnp.float32)

        def zero_row(i, _):
            for j in range(d // 16):
                rows_v[i, pl.ds(j * 16, 16)] = zeros16
            return _

        lax.fori_loop(0, k, zero_row, None)
        for j in range(k // 16):
            ones_v[pl.ds(j * 16, 16)] = ones16

        base = sid * rows_per_sub
        for j in range(rows_per_sub // k):
            pltpu.sync_copy(rows_v, acc_s.at[pl.ds(base + j * k, k)])
        for j in range(rows_per_sub // d):
            pltpu.sync_copy(rows_v.at[0], cnt_s.at[pl.ds(base + j * d, d)])

        pltpu.sync_copy(src_hbm.at[wid], src_v)
        pltpu.sync_copy(dst_hbm.at[wid], dst_v)

        plsc.subcore_barrier()

        def chunk(j, _):
            pltpu.async_copy(table_hbm.at[src_v.at[j]], rows_v, sem).wait()
            pltpu.sync_copy(rows_v, acc_s.at[dst_v.at[j]], add=True)
            pltpu.sync_copy(ones_v, cnt_s.at[dst_v.at[j]], add=True)
            return _

        lax.fori_loop(0, n_chunks, chunk, None)

        plsc.subcore_barrier()

        sl = pl.ds(base, rows_per_sub)

        @pl.when(cid == 0)
        def _():
            pltpu.sync_copy(acc_s.at[sl], psum0.at[sl])
            pltpu.sync_copy(cnt_s.at[sl], pcnt0.at[sl])

        @pl.when(cid == 1)
        def _():
            pltpu.sync_copy(acc_s.at[sl], psum1.at[sl])
            pltpu.sync_copy(cnt_s.at[sl], pcnt1.at[sl])

    return sc_kernel


def _tc_combine(n, d_in, d_out, rb):
    def body(x_ref, p0_ref, p1_ref, c0_ref, c1_ref, w1_ref, w2_ref, b_ref,
             o_ref):
        cnt = jnp.maximum(c0_ref[...] + c1_ref[...], 1.0)
        agg = (p0_ref[...] + p1_ref[...]) / cnt
        o_ref[...] = (
            jnp.dot(x_ref[...], w1_ref[...], preferred_element_type=jnp.float32)
            + jnp.dot(agg, w2_ref[...], preferred_element_type=jnp.float32)
            + b_ref[...]
        )

    grid = (n // rb,)
    return pl.pallas_call(
        body,
        grid=grid,
        in_specs=[
            pl.BlockSpec((rb, d_in), lambda i: (i, 0)),
            pl.BlockSpec((rb, d_in), lambda i: (i, 0)),
            pl.BlockSpec((rb, d_in), lambda i: (i, 0)),
            pl.BlockSpec((rb, 1), lambda i: (i, 0)),
            pl.BlockSpec((rb, 1), lambda i: (i, 0)),
            pl.BlockSpec((d_in, d_out), lambda i: (0, 0)),
            pl.BlockSpec((d_in, d_out), lambda i: (0, 0)),
            pl.BlockSpec((1, d_out), lambda i: (0, 0)),
        ],
        out_specs=pl.BlockSpec((rb, d_out), lambda i: (i, 0)),
        out_shape=jax.ShapeDtypeStruct((n, d_out), jnp.float32),
    )


@jax.jit
def kernel(node_features, edge_indices, W, b):
    bsz, n, d = node_features.shape
    e = edge_indices.shape[-1]
    d_out = W.shape[0]

    nw = 32
    k = 128
    per_w = -(-e // (nw * k)) * k
    n_chunks = per_w // k
    ep = per_w * nw
    n_pad = -(-(n + 64) // (16 * k)) * (16 * k)

    rb = 1
    for cand in range(8, min(n, 1024) + 1, 8):
        if n % cand == 0:
            rb = cand
    if rb == 1:
        rb = n

    sc_fn = _sc_segment_sum(n_pad, d, nw, n_chunks, k)
    tc_fn = _tc_combine(n, d, d_out, rb)

    w1t = jnp.transpose(W[:, :d])
    w2t = jnp.transpose(W[:, d:])
    b2 = b.reshape(1, d_out)

    outs = []
    for bi in range(bsz):
        src = edge_indices[bi, 0, :]
        dst = edge_indices[bi, 1, :]
        pad = ep - e
        pad_ar = jnp.arange(pad, dtype=jnp.int32)
        src_p = jnp.concatenate([src, pad_ar % n]).reshape(nw, n_chunks, k)
        dst_p = jnp.concatenate(
            [dst, n + pad_ar % (n_pad - n)]).reshape(nw, n_chunks, k)

        x = node_features[bi]
        psum0, psum1, pcnt0, pcnt1 = sc_fn(src_p, dst_p, x)
        out = tc_fn(x, psum0, psum1,
                    pcnt0.reshape(n_pad, 1), pcnt1.reshape(n_pad, 1),
                    w1t, w2t, b2)
        outs.append(out)
    return jnp.stack(outs, axis=0)


# speedup vs baseline: 15.7004x; 1.2231x over previous
import functools

import jax
import jax.numpy as jnp
from jax import lax
from jax.experimental import pallas as pl
from jax.experimental.pallas import tpu as pltpu
from jax.experimental.pallas import tpu_sc as plsc


def _sc_segment_sum(n_pad, d, nw, n_chunks, k):
    mesh = plsc.VectorSubcoreMesh(core_axis_name="c", subcore_axis_name="s")
    rows_per_sub = n_pad // 16

    @functools.partial(
        pl.kernel,
        out_type=[
            jax.ShapeDtypeStruct((n_pad, d), jnp.float32),
            jax.ShapeDtypeStruct((n_pad, d), jnp.float32),
            jax.ShapeDtypeStruct((n_pad,), jnp.float32),
            jax.ShapeDtypeStruct((n_pad,), jnp.float32),
        ],
        mesh=mesh,
        scratch_types=[
            pltpu.VMEM((n_chunks, k), jnp.int32),
            pltpu.VMEM((n_chunks, k), jnp.int32),
            pltpu.VMEM((k, d), jnp.float32),
            pltpu.VMEM((k, d), jnp.float32),
            pltpu.VMEM((k,), jnp.float32),
            pltpu.VMEM_SHARED((n_pad, d), jnp.float32),
            pltpu.VMEM_SHARED((n_pad,), jnp.float32),
            pltpu.SemaphoreType.DMA,
            pltpu.SemaphoreType.DMA,
        ],
    )
    def sc_kernel(src_hbm, dst_hbm, table_hbm,
                  psum0, psum1, pcnt0, pcnt1,
                  src_v, dst_v, rows0, rows1, ones_v, acc_s, cnt_s,
                  sem0, sem1):
        cid = lax.axis_index("c")
        sid = lax.axis_index("s")
        wid = sid * 2 + cid

        zeros16 = jnp.zeros((16,), jnp.float32)
        ones16 = jnp.ones((16,), jnp.float32)

        def zero_row(i, _):
            for j in range(d // 16):
                rows0[i, pl.ds(j * 16, 16)] = zeros16
            return _

        lax.fori_loop(0, k, zero_row, None)
        for j in range(k // 16):
            ones_v[pl.ds(j * 16, 16)] = ones16

        base = sid * rows_per_sub
        for j in range(rows_per_sub // k):
            pltpu.sync_copy(rows0, acc_s.at[pl.ds(base + j * k, k)])
        for j in range(rows_per_sub // d):
            pltpu.sync_copy(rows0.at[0], cnt_s.at[pl.ds(base + j * d, d)])

        pltpu.sync_copy(src_hbm.at[wid], src_v)
        pltpu.sync_copy(dst_hbm.at[wid], dst_v)

        plsc.subcore_barrier()

        n_pairs = n_chunks // 2
        pltpu.async_copy(table_hbm.at[src_v.at[0]], rows0, sem0)

        def pair(i, _):
            j0 = 2 * i
            j1 = j0 + 1
            pltpu.make_async_copy(
                table_hbm.at[src_v.at[j0]], rows0, sem0).wait()
            pltpu.async_copy(table_hbm.at[src_v.at[j1]], rows1, sem1)
            pltpu.sync_copy(rows0, acc_s.at[dst_v.at[j0]], add=True)
            pltpu.sync_copy(ones_v, cnt_s.at[dst_v.at[j0]], add=True)
            pltpu.make_async_copy(
                table_hbm.at[src_v.at[j1]], rows1, sem1).wait()
            j2 = jnp.minimum(j0 + 2, n_chunks - 1)

            @pl.when(i + 1 < n_pairs)
            def _():
                pltpu.async_copy(table_hbm.at[src_v.at[j2]], rows0, sem0)

            pltpu.sync_copy(rows1, acc_s.at[dst_v.at[j1]], add=True)
            pltpu.sync_copy(ones_v, cnt_s.at[dst_v.at[j1]], add=True)
            return _

        lax.fori_loop(0, n_pairs, pair, None)

        plsc.subcore_barrier()

        sl = pl.ds(base, rows_per_sub)

        @pl.when(cid == 0)
        def _():
            pltpu.sync_copy(acc_s.at[sl], psum0.at[sl])
            pltpu.sync_copy(cnt_s.at[sl], pcnt0.at[sl])

        @pl.when(cid == 1)
        def _():
            pltpu.sync_copy(acc_s.at[sl], psum1.at[sl])
            pltpu.sync_copy(cnt_s.at[sl], pcnt1.at[sl])

    return sc_kernel


def _tc_combine(n, d_in, d_out, rb):
    def body(x_ref, p0_ref, p1_ref, c0_ref, c1_ref, w1_ref, w2_ref, b_ref,
             o_ref):
        cnt = jnp.maximum(c0_ref[...] + c1_ref[...], 1.0)
        agg = (p0_ref[...] + p1_ref[...]) / cnt
        o_ref[...] = (
            jnp.dot(x_ref[...], w1_ref[...], preferred_element_type=jnp.float32)
            + jnp.dot(agg, w2_ref[...], preferred_element_type=jnp.float32)
            + b_ref[...]
        )

    grid = (n // rb,)
    return pl.pallas_call(
        body,
        grid=grid,
        in_specs=[
            pl.BlockSpec((rb, d_in), lambda i: (i, 0)),
            pl.BlockSpec((rb, d_in), lambda i: (i, 0)),
            pl.BlockSpec((rb, d_in), lambda i: (i, 0)),
            pl.BlockSpec((rb, 1), lambda i: (i, 0)),
            pl.BlockSpec((rb, 1), lambda i: (i, 0)),
            pl.BlockSpec((d_in, d_out), lambda i: (0, 0)),
            pl.BlockSpec((d_in, d_out), lambda i: (0, 0)),
            pl.BlockSpec((1, d_out), lambda i: (0, 0)),
        ],
        out_specs=pl.BlockSpec((rb, d_out), lambda i: (i, 0)),
        out_shape=jax.ShapeDtypeStruct((n, d_out), jnp.float32),
    )


@jax.jit
def kernel(node_features, edge_indices, W, b):
    bsz, n, d = node_features.shape
    e = edge_indices.shape[-1]
    d_out = W.shape[0]

    nw = 32
    k = 128
    per_w = -(-e // (nw * 2 * k)) * (2 * k)
    n_chunks = per_w // k
    ep = per_w * nw
    n_pad = -(-(n + 64) // (16 * k)) * (16 * k)

    rb = 1
    for cand in range(8, min(n, 1024) + 1, 8):
        if n % cand == 0:
            rb = cand
    if rb == 1:
        rb = n

    sc_fn = _sc_segment_sum(n_pad, d, nw, n_chunks, k)
    tc_fn = _tc_combine(n, d, d_out, rb)

    w1t = jnp.transpose(W[:, :d])
    w2t = jnp.transpose(W[:, d:])
    b2 = b.reshape(1, d_out)

    outs = []
    for bi in range(bsz):
        src = edge_indices[bi, 0, :]
        dst = edge_indices[bi, 1, :]
        pad = ep - e
        pad_ar = jnp.arange(pad, dtype=jnp.int32)
        src_p = jnp.concatenate([src, pad_ar % n]).reshape(nw, n_chunks, k)
        dst_p = jnp.concatenate(
            [dst, n + pad_ar % (n_pad - n)]).reshape(nw, n_chunks, k)

        x = node_features[bi]
        psum0, psum1, pcnt0, pcnt1 = sc_fn(src_p, dst_p, x)
        out = tc_fn(x, psum0, psum1,
                    pcnt0.reshape(n_pad, 1), pcnt1.reshape(n_pad, 1),
                    w1t, w2t, b2)
        outs.append(out)
    return jnp.stack(outs, axis=0)


# speedup vs baseline: 15.7226x; 1.0014x over previous
import functools

import jax
import jax.numpy as jnp
from jax import lax
from jax.experimental import pallas as pl
from jax.experimental.pallas import tpu as pltpu
from jax.experimental.pallas import tpu_sc as plsc


def _sc_segment_sum(n_pad, d, nw, n_chunks, k):
    mesh = plsc.VectorSubcoreMesh(core_axis_name="c", subcore_axis_name="s")
    rows_per_sub = n_pad // 16

    @functools.partial(
        pl.kernel,
        out_type=[
            jax.ShapeDtypeStruct((n_pad, d), jnp.float32),
            jax.ShapeDtypeStruct((n_pad, d), jnp.float32),
            jax.ShapeDtypeStruct((n_pad,), jnp.float32),
            jax.ShapeDtypeStruct((n_pad,), jnp.float32),
        ],
        mesh=mesh,
        scratch_types=[
            pltpu.VMEM((n_chunks, k), jnp.int32),
            pltpu.VMEM((n_chunks, k), jnp.int32),
            pltpu.VMEM((k, d), jnp.float32),
            pltpu.VMEM((k, d), jnp.float32),
            pltpu.VMEM((k,), jnp.float32),
            pltpu.VMEM_SHARED((n_pad, d), jnp.float32),
            pltpu.VMEM_SHARED((n_pad,), jnp.float32),
            pltpu.SemaphoreType.DMA,
            pltpu.SemaphoreType.DMA,
            pltpu.SemaphoreType.DMA,
        ],
    )
    def sc_kernel(src_hbm, dst_hbm, table_hbm,
                  psum0, psum1, pcnt0, pcnt1,
                  src_v, dst_v, rows0, rows1, ones_v, acc_s, cnt_s,
                  sem0, sem1, semc):
        cid = lax.axis_index("c")
        sid = lax.axis_index("s")
        wid = sid * 2 + cid

        zeros16 = jnp.zeros((16,), jnp.float32)
        ones16 = jnp.ones((16,), jnp.float32)

        def zero_row(i, _):
            for j in range(d // 16):
                rows0[i, pl.ds(j * 16, 16)] = zeros16
            return _

        lax.fori_loop(0, k, zero_row, None)
        for j in range(k // 16):
            ones_v[pl.ds(j * 16, 16)] = ones16

        base = sid * rows_per_sub
        for j in range(rows_per_sub // k):
            pltpu.sync_copy(rows0, acc_s.at[pl.ds(base + j * k, k)])
        for j in range(rows_per_sub // d):
            pltpu.sync_copy(rows0.at[0], cnt_s.at[pl.ds(base + j * d, d)])

        pltpu.sync_copy(src_hbm.at[wid], src_v)
        pltpu.sync_copy(dst_hbm.at[wid], dst_v)

        plsc.subcore_barrier()

        n_pairs = n_chunks // 2
        pltpu.async_copy(table_hbm.at[src_v.at[0]], rows0, sem0)

        def pair(i, _):
            j0 = 2 * i
            j1 = j0 + 1
            pltpu.make_async_copy(
                table_hbm.at[src_v.at[j0]], rows0, sem0).wait()
            pltpu.async_copy(table_hbm.at[src_v.at[j1]], rows1, sem1)
            pltpu.async_copy(ones_v, cnt_s.at[dst_v.at[j0]], semc, add=True)
            pltpu.sync_copy(rows0, acc_s.at[dst_v.at[j0]], add=True)
            pltpu.make_async_copy(
                table_hbm.at[src_v.at[j1]], rows1, sem1).wait()
            j2 = jnp.minimum(j0 + 2, n_chunks - 1)

            @pl.when(i + 1 < n_pairs)
            def _():
                pltpu.async_copy(table_hbm.at[src_v.at[j2]], rows0, sem0)

            pltpu.async_copy(ones_v, cnt_s.at[dst_v.at[j1]], semc, add=True)
            pltpu.sync_copy(rows1, acc_s.at[dst_v.at[j1]], add=True)
            return _

        lax.fori_loop(0, n_pairs, pair, None)

        def drain(i, _):
            pltpu.make_async_copy(
                ones_v, cnt_s.at[dst_v.at[i]], semc).wait()
            return _

        lax.fori_loop(0, n_chunks, drain, None)

        plsc.subcore_barrier()

        sl = pl.ds(base, rows_per_sub)

        @pl.when(cid == 0)
        def _():
            pltpu.sync_copy(acc_s.at[sl], psum0.at[sl])
            pltpu.sync_copy(cnt_s.at[sl], pcnt0.at[sl])

        @pl.when(cid == 1)
        def _():
            pltpu.sync_copy(acc_s.at[sl], psum1.at[sl])
            pltpu.sync_copy(cnt_s.at[sl], pcnt1.at[sl])

    return sc_kernel


def _tc_combine(n, d_in, d_out, rb):
    def body(x_ref, p0_ref, p1_ref, c0_ref, c1_ref, w1_ref, w2_ref, b_ref,
             o_ref):
        cnt = jnp.maximum(c0_ref[...] + c1_ref[...], 1.0)
        agg = (p0_ref[...] + p1_ref[...]) / cnt
        o_ref[...] = (
            jnp.dot(x_ref[...], w1_ref[...], preferred_element_type=jnp.float32)
            + jnp.dot(agg, w2_ref[...], preferred_element_type=jnp.float32)
            + b_ref[...]
        )

    grid = (n // rb,)
    return pl.pallas_call(
        body,
        grid=grid,
        in_specs=[
            pl.BlockSpec((rb, d_in), lambda i: (i, 0)),
            pl.BlockSpec((rb, d_in), lambda i: (i, 0)),
            pl.BlockSpec((rb, d_in), lambda i: (i, 0)),
            pl.BlockSpec((rb, 1), lambda i: (i, 0)),
            pl.BlockSpec((rb, 1), lambda i: (i, 0)),
            pl.BlockSpec((d_in, d_out), lambda i: (0, 0)),
            pl.BlockSpec((d_in, d_out), lambda i: (0, 0)),
            pl.BlockSpec((1, d_out), lambda i: (0, 0)),
        ],
        out_specs=pl.BlockSpec((rb, d_out), lambda i: (i, 0)),
        out_shape=jax.ShapeDtypeStruct((n, d_out), jnp.float32),
    )


@jax.jit
def kernel(node_features, edge_indices, W, b):
    bsz, n, d = node_features.shape
    e = edge_indices.shape[-1]
    d_out = W.shape[0]

    nw = 32
    k = 128
    per_w = -(-e // (nw * 2 * k)) * (2 * k)
    n_chunks = per_w // k
    ep = per_w * nw
    n_pad = -(-(n + 64) // (16 * k)) * (16 * k)

    rb = 1
    for cand in range(8, min(n, 1024) + 1, 8):
        if n % cand == 0:
            rb = cand
    if rb == 1:
        rb = n

    sc_fn = _sc_segment_sum(n_pad, d, nw, n_chunks, k)
    tc_fn = _tc_combine(n, d, d_out, rb)

    w1t = jnp.transpose(W[:, :d])
    w2t = jnp.transpose(W[:, d:])
    b2 = b.reshape(1, d_out)

    outs = []
    for bi in range(bsz):
        src = edge_indices[bi, 0, :]
        dst = edge_indices[bi, 1, :]
        pad = ep - e
        pad_ar = jnp.arange(pad, dtype=jnp.int32)
        src_p = jnp.concatenate([src, pad_ar % n]).reshape(nw, n_chunks, k)
        dst_p = jnp.concatenate(
            [dst, n + pad_ar % (n_pad - n)]).reshape(nw, n_chunks, k)

        x = node_features[bi]
        psum0, psum1, pcnt0, pcnt1 = sc_fn(src_p, dst_p, x)
        out = tc_fn(x, psum0, psum1,
                    pcnt0.reshape(n_pad, 1), pcnt1.reshape(n_pad, 1),
                    w1t, w2t, b2)
        outs.append(out[None])
    return jnp.concatenate(outs, axis=0) if bsz > 1 else outs[0]
